# baseline (speedup 1.0000x reference)
import jax
import jax.numpy as jnp
from jax import lax
from jax.experimental import pallas as pl
from jax.experimental.pallas import tpu as pltpu
from jax.experimental.pallas import tpu_sc as plsc

_B, _S, _D = 4, 4096, 2048
_NC, _NS = 2, 16
_NW = _NC * _NS
_SPW = _S // _NW
_C = 8
_NCHUNK = _SPW // _C
_NITEM = _NCHUNK * _B


def _sc_body(x_hbm, pe_hbm, out_hbm, x_v, pe_v, in_sem, out_sem, pe_sem):
    wid = lax.axis_index("s") * _NC + lax.axis_index("c")
    s0 = wid * _SPW

    def x_slice(k):
        c = k >> 2
        b = k & 3
        return x_hbm.at[b, pl.ds(s0 + c * _C, _C)]

    def out_slice(k):
        c = k >> 2
        b = k & 3
        return out_hbm.at[b, pl.ds(s0 + c * _C, _C)]

    pltpu.async_copy(pe_hbm.at[pl.ds(s0, _C)], pe_v.at[0], pe_sem.at[0])
    pltpu.async_copy(x_slice(0), x_v.at[0], in_sem.at[0])
    pltpu.async_copy(x_slice(1), x_v.at[1], in_sem.at[1])

    def item(k, carry):
        c = k >> 2
        b = k & 3
        r = lax.rem(k, 3)
        cp = c & 1

        @pl.when(b == 0)
        def _():
            @pl.when(c + 1 < _NCHUNK)
            def _():
                pltpu.async_copy(
                    pe_hbm.at[pl.ds(s0 + (c + 1) * _C, _C)],
                    pe_v.at[1 - cp],
                    pe_sem.at[1 - cp],
                )
            pltpu.make_async_copy(
                pe_hbm.at[pl.ds(s0 + c * _C, _C)], pe_v.at[cp], pe_sem.at[cp]
            ).wait()

        pltpu.make_async_copy(x_slice(k), x_v.at[r], in_sem.at[r]).wait()

        @pl.when(k + 2 < _NITEM)
        def _():
            q = lax.rem(k + 2, 3)

            @pl.when(k >= 1)
            def _():
                pltpu.make_async_copy(
                    x_v.at[q], out_slice(k - 1), out_sem.at[q]
                ).wait()

            pltpu.async_copy(x_slice(k + 2), x_v.at[q], in_sem.at[q])

        c_out = out_slice(k)
        for rr in range(_C):

            @plsc.parallel_loop(0, _D, step=16, unroll=8)
            def add(cc):
                cc = pl.multiple_of(cc, 16)
                x_v[r, rr, pl.ds(cc, 16)] = (
                    x_v[r, rr, pl.ds(cc, 16)] + pe_v[cp, rr, pl.ds(cc, 16)]
                )

            pltpu.async_copy(x_v.at[r, rr], c_out.at[rr], out_sem.at[r])
        return carry

    lax.fori_loop(0, _NITEM, item, None)

    for k in (_NITEM - 3, _NITEM - 2, _NITEM - 1):
        r = k % 3
        pltpu.make_async_copy(x_v.at[r], out_slice(k), out_sem.at[r]).wait()


def kernel(x, pe_weight):
    B, S, D = x.shape
    mesh = plsc.VectorSubcoreMesh(core_axis_name="c", subcore_axis_name="s")
    return pl.kernel(
        _sc_body,
        out_type=jax.ShapeDtypeStruct((B, S, D), jnp.float32),
        mesh=mesh,
        scratch_types=[
            pltpu.VMEM((3, _C, _D), jnp.float32),
            pltpu.VMEM((2, _C, _D), jnp.float32),
            pltpu.SemaphoreType.DMA((3,)),
            pltpu.SemaphoreType.DMA((3,)),
            pltpu.SemaphoreType.DMA((2,)),
        ],
        compiler_params=pltpu.CompilerParams(use_tc_tiling_on_sc=True),
    )(x, pe_weight)


# speedup vs baseline: 1.0134x; 1.0134x over previous
import jax
import jax.numpy as jnp
from jax import lax
from jax.experimental import pallas as pl
from jax.experimental.pallas import tpu as pltpu
from jax.experimental.pallas import tpu_sc as plsc

_B, _S, _D = 4, 4096, 2048
_NC, _NS = 2, 16
_NW = _NC * _NS
_SPW = _S // _NW
_C = 8
_NCHUNK = _SPW // _C
_NITEM = _NCHUNK * _B


def _sc_body(x_hbm, pe_hbm, out_hbm, x_v, pe_v, in_sem, out_sem, pe_sem):
    wid = lax.axis_index("s") * _NC + lax.axis_index("c")
    s0 = wid * _SPW

    def x_slice(k):
        c = k >> 2
        b = k & 3
        return x_hbm.at[b, pl.ds(s0 + c * _C, _C)]

    def out_slice(k):
        c = k >> 2
        b = k & 3
        return out_hbm.at[b, pl.ds(s0 + c * _C, _C)]

    pltpu.async_copy(pe_hbm.at[pl.ds(s0, _C)], pe_v.at[0], pe_sem.at[0])
    pltpu.async_copy(x_slice(0), x_v.at[0], in_sem.at[0])
    pltpu.async_copy(x_slice(1), x_v.at[1], in_sem.at[1])
    pltpu.async_copy(x_slice(2), x_v.at[2], in_sem.at[2])

    def item(k, carry):
        c = k >> 2
        b = k & 3
        r = k & 3
        cp = c & 1

        @pl.when(b == 0)
        def _():
            @pl.when(c + 1 < _NCHUNK)
            def _():
                pltpu.async_copy(
                    pe_hbm.at[pl.ds(s0 + (c + 1) * _C, _C)],
                    pe_v.at[1 - cp],
                    pe_sem.at[1 - cp],
                )
            pltpu.make_async_copy(
                pe_hbm.at[pl.ds(s0 + c * _C, _C)], pe_v.at[cp], pe_sem.at[cp]
            ).wait()

        pltpu.make_async_copy(x_slice(k), x_v.at[r], in_sem.at[r]).wait()

        @pl.when(k + 3 < _NITEM)
        def _():
            q = (k + 3) & 3

            @pl.when(k >= 1)
            def _():
                pltpu.make_async_copy(
                    x_v.at[q], out_slice(k - 1), out_sem.at[q]
                ).wait()

            pltpu.async_copy(x_slice(k + 3), x_v.at[q], in_sem.at[q])

        c_out = out_slice(k)
        for rr in range(_C):

            @plsc.parallel_loop(0, _D, step=16, unroll=8)
            def add(cc):
                cc = pl.multiple_of(cc, 16)
                x_v[r, rr, pl.ds(cc, 16)] = (
                    x_v[r, rr, pl.ds(cc, 16)] + pe_v[cp, rr, pl.ds(cc, 16)]
                )

            pltpu.async_copy(x_v.at[r, rr], c_out.at[rr], out_sem.at[r])
        return carry

    lax.fori_loop(0, _NITEM, item, None)

    for k in (_NITEM - 4, _NITEM - 3, _NITEM - 2, _NITEM - 1):
        r = k & 3
        pltpu.make_async_copy(x_v.at[r], out_slice(k), out_sem.at[r]).wait()


def kernel(x, pe_weight):
    B, S, D = x.shape
    mesh = plsc.VectorSubcoreMesh(core_axis_name="c", subcore_axis_name="s")
    return pl.kernel(
        _sc_body,
        out_type=jax.ShapeDtypeStruct((B, S, D), jnp.float32),
        mesh=mesh,
        scratch_types=[
            pltpu.VMEM((4, _C, _D), jnp.float32),
            pltpu.VMEM((2, _C, _D), jnp.float32),
            pltpu.SemaphoreType.DMA((4,)),
            pltpu.SemaphoreType.DMA((4,)),
            pltpu.SemaphoreType.DMA((2,)),
        ],
        compiler_params=pltpu.CompilerParams(use_tc_tiling_on_sc=True),
    )(x, pe_weight)
